# Initial kernel scaffold; baseline (speedup 1.0000x reference)
import jax
import jax.numpy as jnp
from jax.experimental import pallas as pl


def kernel(ndata_weight, edge_index, W0, b0, W1, b1):
    raise NotImplementedError("write your pallas kernel here")


# speedup vs baseline: 7.8495x; 7.8495x over previous
import functools

import jax
import jax.numpy as jnp
from jax import lax
from jax.experimental import pallas as pl
from jax.experimental.pallas import tpu as pltpu
from jax.experimental.pallas import tpu_sc as plsc

N = 10000
E = 160000
D = 256
H = 128
CH = 80
EPT = E // 16
NCH = EPT // CH
RPT = 624
TAIL = N - 16 * RPT
ZR = 156

_MESH = plsc.VectorSubcoreMesh(
    core_axis_name="c", subcore_axis_name="s", num_cores=2, num_subcores=16)


@functools.partial(
    pl.kernel,
    out_type=jax.ShapeDtypeStruct((2 * N, 16), jnp.float32),
    mesh=_MESH,
    scratch_types=[
        pltpu.VMEM((CH,), jnp.int32),
        pltpu.VMEM((CH, 16), jnp.float32),
        pltpu.VMEM((ZR, 16), jnp.float32),
        pltpu.VMEM_SHARED((N, 16), jnp.float32),
    ],
)
def _sc_degree(dst_hbm, out_hbm, dst_v, ones_v, zb, acc):
    c = lax.axis_index("c")
    s = lax.axis_index("s")

    def _fill_ones(i, carry):
        ones_v[i] = jnp.ones((16,), jnp.float32)
        return carry

    lax.fori_loop(0, CH, _fill_ones, 0)

    def _fill_zero(i, carry):
        zb[i] = jnp.zeros((16,), jnp.float32)
        return carry

    lax.fori_loop(0, ZR, _fill_zero, 0)

    row0 = pl.multiple_of(s * RPT, 8)
    for k in range(RPT // ZR):
        pltpu.sync_copy(zb, acc.at[pl.ds(row0 + k * ZR, ZR)])

    @pl.when(s == 15)
    def _zero_tail():
        pltpu.sync_copy(zb.at[pl.ds(0, TAIL)], acc.at[pl.ds(N - TAIL, TAIL)])

    plsc.subcore_barrier()

    base = s * EPT

    def _chunk(k, carry):
        off = pl.multiple_of(base + k * CH, 8)
        pltpu.sync_copy(dst_hbm.at[pl.ds(off, CH)], dst_v)
        pltpu.sync_copy(ones_v, acc.at[dst_v], add=True)
        return carry

    lax.fori_loop(0, NCH, _chunk, 0)
    plsc.subcore_barrier()
    pltpu.sync_copy(acc.at[pl.ds(row0, RPT)],
                    out_hbm.at[pl.ds(pl.multiple_of(c * N + s * RPT, 8), RPT)])

    @pl.when(s == 15)
    def _copy_tail():
        pltpu.sync_copy(
            acc.at[pl.ds(N - TAIL, TAIL)],
            out_hbm.at[pl.ds(pl.multiple_of(c * N + N - TAIL, 8), TAIL)])


@functools.partial(
    pl.kernel,
    out_type=jax.ShapeDtypeStruct((2 * N, H), jnp.float32),
    mesh=_MESH,
    scratch_types=[
        pltpu.VMEM((CH,), jnp.int32),
        pltpu.VMEM((CH,), jnp.int32),
        pltpu.VMEM((CH, H), jnp.float32),
        pltpu.VMEM((ZR, H), jnp.float32),
        pltpu.VMEM_SHARED((N, H), jnp.float32),
        pltpu.SemaphoreType.DMA,
    ],
)
def _sc_aggregate(y_hbm, src2_hbm, dst_hbm, out_hbm,
                  idx_v, dst_v, rows_v, zb, acc, sem):
    c = lax.axis_index("c")
    s = lax.axis_index("s")

    def _fill_zero(i, carry):
        for j in range(H // 16):
            zb[i, pl.ds(j * 16, 16)] = jnp.zeros((16,), jnp.float32)
        return carry

    lax.fori_loop(0, ZR, _fill_zero, 0)

    row0 = pl.multiple_of(s * RPT, 8)
    for k in range(RPT // ZR):
        pltpu.sync_copy(zb, acc.at[pl.ds(row0 + k * ZR, ZR)])

    @pl.when(s == 15)
    def _zero_tail():
        pltpu.sync_copy(zb.at[pl.ds(0, TAIL)], acc.at[pl.ds(N - TAIL, TAIL)])

    plsc.subcore_barrier()

    ebase = c * E + s * EPT
    dbase = s * EPT

    def _chunk(k, carry):
        eoff = pl.multiple_of(ebase + k * CH, 8)
        doff = pl.multiple_of(dbase + k * CH, 8)
        pltpu.sync_copy(src2_hbm.at[pl.ds(eoff, CH)], idx_v)
        pltpu.sync_copy(dst_hbm.at[pl.ds(doff, CH)], dst_v)
        pltpu.async_copy(y_hbm.at[idx_v], rows_v, sem).wait()
        pltpu.sync_copy(rows_v, acc.at[dst_v], add=True)
        return carry

    lax.fori_loop(0, NCH, _chunk, 0)
    plsc.subcore_barrier()
    pltpu.sync_copy(acc.at[pl.ds(row0, RPT)],
                    out_hbm.at[pl.ds(pl.multiple_of(c * N + s * RPT, 8), RPT)])

    @pl.when(s == 15)
    def _copy_tail():
        pltpu.sync_copy(
            acc.at[pl.ds(N - TAIL, TAIL)],
            out_hbm.at[pl.ds(pl.multiple_of(c * N + N - TAIL, 8), TAIL)])


_BN = 2000
_NB = N // _BN


def _dinv_of(deg_ref):
    return lax.rsqrt(deg_ref[:, 0:1] + 1.0)


def _tc1_body(x_ref, w_ref, deg_ref, out_ref):
    dinv = _dinv_of(deg_ref)
    xw = jnp.dot(x_ref[...], w_ref[...],
                 preferred_element_type=jnp.float32,
                 precision=lax.Precision.HIGHEST)
    y = xw * dinv
    out_ref[0] = y[:, :H]
    out_ref[1] = y[:, H:]


def _tc1(x, w0, deg2):
    return pl.pallas_call(
        _tc1_body,
        grid=(_NB,),
        in_specs=[
            pl.BlockSpec((_BN, D), lambda i: (i, 0)),
            pl.BlockSpec((D, D), lambda i: (0, 0)),
            pl.BlockSpec((_BN, 16), lambda i: (i, 0)),
        ],
        out_specs=pl.BlockSpec((2, _BN, H), lambda i: (0, i, 0)),
        out_shape=jax.ShapeDtypeStruct((2, N, H), jnp.float32),
    )(x, w0, deg2)


def _tc2_body(agg_a, agg_b, y_a, y_b, deg_ref, b0_ref, w_ref, out_ref):
    dinv = _dinv_of(deg_ref)
    ha = jnp.maximum(dinv * (agg_a[...] + y_a[...]) + b0_ref[0:1, :], 0.0)
    hb = jnp.maximum(dinv * (agg_b[...] + y_b[...]) + b0_ref[1:2, :], 0.0)
    h = jnp.concatenate([ha, hb], axis=1)
    xw = jnp.dot(h, w_ref[...],
                 preferred_element_type=jnp.float32,
                 precision=lax.Precision.HIGHEST)
    y = xw * dinv
    out_ref[0] = y[:, :H]
    out_ref[1] = y[:, H:]


def _tc2(agg1, y1, deg2, b0r, w1):
    half_a = pl.BlockSpec((_BN, H), lambda i: (i, 0))
    half_b = pl.BlockSpec((_BN, H), lambda i: (_NB + i, 0))
    return pl.pallas_call(
        _tc2_body,
        grid=(_NB,),
        in_specs=[
            half_a, half_b, half_a, half_b,
            pl.BlockSpec((_BN, 16), lambda i: (i, 0)),
            pl.BlockSpec((2, H), lambda i: (0, 0)),
            pl.BlockSpec((D, D), lambda i: (0, 0)),
        ],
        out_specs=pl.BlockSpec((2, _BN, H), lambda i: (0, i, 0)),
        out_shape=jax.ShapeDtypeStruct((2, N, H), jnp.float32),
    )(agg1, agg1, y1, y1, deg2, b0r, w1)


def _tc3_body(agg_a, agg_b, y_a, y_b, deg_ref, b1_ref, out_ref):
    dinv = _dinv_of(deg_ref)
    za = dinv * (agg_a[...] + y_a[...])
    zb = dinv * (agg_b[...] + y_b[...])
    out_ref[...] = jnp.concatenate([za, zb], axis=1) + b1_ref[...]


def _tc3(agg2, y2, deg2, b1r):
    half_a = pl.BlockSpec((_BN, H), lambda i: (i, 0))
    half_b = pl.BlockSpec((_BN, H), lambda i: (_NB + i, 0))
    return pl.pallas_call(
        _tc3_body,
        grid=(_NB,),
        in_specs=[
            half_a, half_b, half_a, half_b,
            pl.BlockSpec((_BN, 16), lambda i: (i, 0)),
            pl.BlockSpec((1, D), lambda i: (0, 0)),
        ],
        out_specs=pl.BlockSpec((_BN, D), lambda i: (i, 0)),
        out_shape=jax.ShapeDtypeStruct((N, D), jnp.float32),
    )(agg2, agg2, y2, y2, deg2, b1r)


def kernel(ndata_weight, edge_index, W0, b0, W1, b1):
    src = edge_index[0]
    dst = edge_index[1]
    src2 = jnp.concatenate([src, src + N], axis=0)
    b0r = b0.reshape(2, H)
    b1r = b1.reshape(1, D)

    deg2 = _sc_degree(dst)
    y1 = _tc1(ndata_weight, W0, deg2).reshape(2 * N, H)
    agg1 = _sc_aggregate(y1, src2, dst)
    y2 = _tc2(agg1, y1, deg2, b0r, W1).reshape(2 * N, H)
    agg2 = _sc_aggregate(y2, src2, dst)
    return _tc3(agg2, y2, deg2, b1r)


# speedup vs baseline: 13.1145x; 1.6707x over previous
import functools

import jax
import jax.numpy as jnp
from jax import lax
from jax.experimental import pallas as pl
from jax.experimental.pallas import tpu as pltpu
from jax.experimental.pallas import tpu_sc as plsc

N = 10000
E = 160000
D = 256
H = 128
CH = 80
EPT = E // 16
NCH = EPT // CH
CHD = 125
NCHD = EPT // CHD
RPT = 624
TAIL = N - 16 * RPT
ZR = 156

_MESH = plsc.VectorSubcoreMesh(
    core_axis_name="c", subcore_axis_name="s", num_cores=2, num_subcores=16)


@functools.partial(
    pl.kernel,
    out_type=jax.ShapeDtypeStruct((2 * N, 16), jnp.float32),
    mesh=_MESH,
    scratch_types=[
        pltpu.VMEM((NCHD, CHD), jnp.int32),
        pltpu.VMEM((CHD, 16), jnp.float32),
        pltpu.VMEM((ZR, 16), jnp.float32),
        pltpu.VMEM_SHARED((N, 16), jnp.float32),
    ],
)
def _sc_degree(dst_hbm, out_hbm, didx, ones_v, zb, acc):
    c = lax.axis_index("c")
    s = lax.axis_index("s")

    pltpu.sync_copy(dst_hbm.at[pl.ds(pl.multiple_of(s * NCHD, 8), NCHD)], didx)

    def _fill_ones(i, carry):
        ones_v[i] = jnp.ones((16,), jnp.float32)
        return carry

    lax.fori_loop(0, CHD, _fill_ones, 0)

    def _fill_zero(i, carry):
        zb[i] = jnp.zeros((16,), jnp.float32)
        return carry

    lax.fori_loop(0, ZR, _fill_zero, 0)

    row0 = pl.multiple_of(s * RPT, 8)
    for k in range(RPT // ZR):
        pltpu.sync_copy(zb, acc.at[pl.ds(row0 + k * ZR, ZR)])

    @pl.when(s == 15)
    def _zero_tail():
        pltpu.sync_copy(zb.at[pl.ds(0, TAIL)], acc.at[pl.ds(N - TAIL, TAIL)])

    plsc.subcore_barrier()

    def _chunk(k, carry):
        pltpu.sync_copy(ones_v, acc.at[didx.at[k]], add=True)
        return carry

    lax.fori_loop(0, NCHD, _chunk, 0)
    plsc.subcore_barrier()
    pltpu.sync_copy(acc.at[pl.ds(row0, RPT)],
                    out_hbm.at[pl.ds(pl.multiple_of(c * N + s * RPT, 8), RPT)])

    @pl.when(s == 15)
    def _copy_tail():
        pltpu.sync_copy(
            acc.at[pl.ds(N - TAIL, TAIL)],
            out_hbm.at[pl.ds(pl.multiple_of(c * N + N - TAIL, 8), TAIL)])


@functools.partial(
    pl.kernel,
    out_type=jax.ShapeDtypeStruct((2 * N, H), jnp.float32),
    mesh=_MESH,
    scratch_types=[
        pltpu.VMEM((CH,), jnp.int32),
        pltpu.VMEM((CH,), jnp.int32),
        pltpu.VMEM((CH,), jnp.int32),
        pltpu.VMEM((CH,), jnp.int32),
        pltpu.VMEM((CH, H), jnp.float32),
        pltpu.VMEM((CH, H), jnp.float32),
        pltpu.VMEM((ZR, H), jnp.float32),
        pltpu.VMEM_SHARED((N, H), jnp.float32),
        pltpu.SemaphoreType.DMA,
    ],
)
def _sc_aggregate(y_hbm, src2_hbm, dst_hbm, out_hbm,
                  sidx_a, sidx_b, didx_a, didx_b, rows_a, rows_b,
                  zb, acc, sem_g):
    c = lax.axis_index("c")
    s = lax.axis_index("s")
    ebase = c * E + s * EPT
    dbase = s * EPT

    def _fetch(k, si, di):
        pltpu.sync_copy(
            src2_hbm.at[pl.ds(pl.multiple_of(ebase + k * CH, 8), CH)], si)
        pltpu.sync_copy(
            dst_hbm.at[pl.ds(pl.multiple_of(dbase + k * CH, 8), CH)], di)

    def _gather(si, rv):
        pltpu.async_copy(y_hbm.at[si], rv, sem_g)

    def _gwait(si, rv):
        pltpu.make_async_copy(y_hbm.at[si], rv, sem_g).wait()

    def _scat(rv, di):
        pltpu.sync_copy(rv, acc.at[di], add=True)

    def _fill_zero(i, carry):
        for j in range(H // 16):
            zb[i, pl.ds(j * 16, 16)] = jnp.zeros((16,), jnp.float32)
        return carry

    lax.fori_loop(0, ZR, _fill_zero, 0)

    row0 = pl.multiple_of(s * RPT, 8)
    for k in range(RPT // ZR):
        pltpu.sync_copy(zb, acc.at[pl.ds(row0 + k * ZR, ZR)])

    @pl.when(s == 15)
    def _zero_tail():
        pltpu.sync_copy(zb.at[pl.ds(0, TAIL)], acc.at[pl.ds(N - TAIL, TAIL)])

    _fetch(0, sidx_a, didx_a)
    _gather(sidx_a, rows_a)
    plsc.subcore_barrier()

    def _pair(i, carry):
        a = 2 * i
        _fetch(a + 1, sidx_b, didx_b)
        _gwait(sidx_a, rows_a)
        _gather(sidx_b, rows_b)
        _scat(rows_a, didx_a)
        _fetch(a + 2, sidx_a, didx_a)
        _gwait(sidx_b, rows_b)
        _gather(sidx_a, rows_a)
        _scat(rows_b, didx_b)
        return carry

    lax.fori_loop(0, (NCH - 1) // 2, _pair, 0)
    _gwait(sidx_a, rows_a)
    _scat(rows_a, didx_a)
    plsc.subcore_barrier()
    pltpu.sync_copy(acc.at[pl.ds(row0, RPT)],
                    out_hbm.at[pl.ds(pl.multiple_of(c * N + s * RPT, 8), RPT)])

    @pl.when(s == 15)
    def _copy_tail():
        pltpu.sync_copy(
            acc.at[pl.ds(N - TAIL, TAIL)],
            out_hbm.at[pl.ds(pl.multiple_of(c * N + N - TAIL, 8), TAIL)])


_BN = 2000
_NB = N // _BN


def _dinv_of(deg_ref):
    return lax.rsqrt(deg_ref[:, 0:1] + 1.0)


def _tc1_body(x_ref, w_ref, deg_ref, out_ref):
    dinv = _dinv_of(deg_ref)
    xw = jnp.dot(x_ref[...], w_ref[...],
                 preferred_element_type=jnp.float32,
                 precision=lax.Precision.HIGHEST)
    y = xw * dinv
    out_ref[0] = y[:, :H]
    out_ref[1] = y[:, H:]


def _tc1(x, w0, deg2):
    return pl.pallas_call(
        _tc1_body,
        grid=(_NB,),
        in_specs=[
            pl.BlockSpec((_BN, D), lambda i: (i, 0)),
            pl.BlockSpec((D, D), lambda i: (0, 0)),
            pl.BlockSpec((_BN, 16), lambda i: (i, 0)),
        ],
        out_specs=pl.BlockSpec((2, _BN, H), lambda i: (0, i, 0)),
        out_shape=jax.ShapeDtypeStruct((2, N, H), jnp.float32),
    )(x, w0, deg2)


def _tc2_body(agg_a, agg_b, y_a, y_b, deg_ref, b0_ref, w_ref, out_ref):
    dinv = _dinv_of(deg_ref)
    ha = jnp.maximum(dinv * (agg_a[...] + y_a[...]) + b0_ref[0:1, :], 0.0)
    hb = jnp.maximum(dinv * (agg_b[...] + y_b[...]) + b0_ref[1:2, :], 0.0)
    h = jnp.concatenate([ha, hb], axis=1)
    xw = jnp.dot(h, w_ref[...],
                 preferred_element_type=jnp.float32,
                 precision=lax.Precision.HIGHEST)
    y = xw * dinv
    out_ref[0] = y[:, :H]
    out_ref[1] = y[:, H:]


def _tc2(agg1, y1, deg2, b0r, w1):
    half_a = pl.BlockSpec((_BN, H), lambda i: (i, 0))
    half_b = pl.BlockSpec((_BN, H), lambda i: (_NB + i, 0))
    return pl.pallas_call(
        _tc2_body,
        grid=(_NB,),
        in_specs=[
            half_a, half_b, half_a, half_b,
            pl.BlockSpec((_BN, 16), lambda i: (i, 0)),
            pl.BlockSpec((2, H), lambda i: (0, 0)),
            pl.BlockSpec((D, D), lambda i: (0, 0)),
        ],
        out_specs=pl.BlockSpec((2, _BN, H), lambda i: (0, i, 0)),
        out_shape=jax.ShapeDtypeStruct((2, N, H), jnp.float32),
    )(agg1, agg1, y1, y1, deg2, b0r, w1)


def _tc3_body(agg_a, agg_b, y_a, y_b, deg_ref, b1_ref, out_ref):
    dinv = _dinv_of(deg_ref)
    za = dinv * (agg_a[...] + y_a[...])
    zb = dinv * (agg_b[...] + y_b[...])
    out_ref[...] = jnp.concatenate([za, zb], axis=1) + b1_ref[...]


def _tc3(agg2, y2, deg2, b1r):
    half_a = pl.BlockSpec((_BN, H), lambda i: (i, 0))
    half_b = pl.BlockSpec((_BN, H), lambda i: (_NB + i, 0))
    return pl.pallas_call(
        _tc3_body,
        grid=(_NB,),
        in_specs=[
            half_a, half_b, half_a, half_b,
            pl.BlockSpec((_BN, 16), lambda i: (i, 0)),
            pl.BlockSpec((1, D), lambda i: (0, 0)),
        ],
        out_specs=pl.BlockSpec((_BN, D), lambda i: (i, 0)),
        out_shape=jax.ShapeDtypeStruct((N, D), jnp.float32),
    )(agg2, agg2, y2, y2, deg2, b1r)


def kernel(ndata_weight, edge_index, W0, b0, W1, b1):
    src = edge_index[0]
    dst = edge_index[1]
    src2 = jnp.concatenate([src, src + N], axis=0)
    dst2 = dst.reshape(E // CHD, CHD)
    b0r = b0.reshape(2, H)
    b1r = b1.reshape(1, D)

    deg2 = _sc_degree(dst2)
    y1 = _tc1(ndata_weight, W0, deg2).reshape(2 * N, H)
    agg1 = _sc_aggregate(y1, src2, dst)
    y2 = _tc2(agg1, y1, deg2, b0r, W1).reshape(2 * N, H)
    agg2 = _sc_aggregate(y2, src2, dst)
    return _tc3(agg2, y2, deg2, b1r)
